# baseline (speedup 1.0000x reference)
import functools

import jax
import jax.numpy as jnp
from jax import lax
from jax.experimental import pallas as pl
from jax.experimental.pallas import tpu as pltpu
from jax.experimental.pallas import tpu_sc as plsc

BATCH = 4
SEQ = 8192
D = 1024

NC = 2
NS = 16
NW = NC * NS
B_PER_W = BATCH * SEQ // NW
W_PER_BATCH = SEQ // B_PER_W
C = 32
NCHUNK = B_PER_W // C
NB = 2


def _sc_gather(weight, input_ids):
  mesh = plsc.VectorSubcoreMesh(core_axis_name="c", subcore_axis_name="s")

  @functools.partial(
      pl.kernel,
      mesh=mesh,
      out_type=jax.ShapeDtypeStruct((BATCH, SEQ, D), jnp.float32),
      scratch_types=[
          pltpu.VMEM((B_PER_W,), jnp.int32),
          pltpu.VMEM((NB, C, D), jnp.float32),
          pltpu.SemaphoreType.DMA((NB,)),
      ],
      compiler_params=pltpu.CompilerParams(skip_device_barrier=True),
  )
  def k(table_hbm, idx_hbm, out_hbm, idx_v, rows_v, gsem):
    wid = lax.axis_index("s") * NC + lax.axis_index("c")
    bb = wid // W_PER_BATCH
    col = (wid % W_PER_BATCH) * B_PER_W
    pltpu.sync_copy(idx_hbm.at[bb, pl.ds(col, B_PER_W)], idx_v)

    def start_gather(chunk, b):
      pltpu.async_copy(
          table_hbm.at[idx_v.at[pl.ds(chunk * C, C)]], rows_v.at[b], gsem.at[b]
      )

    def wait_gather(chunk, b):
      pltpu.make_async_copy(
          table_hbm.at[idx_v.at[pl.ds(chunk * C, C)]], rows_v.at[b], gsem.at[b]
      ).wait()

    def put(chunk, b):
      pltpu.sync_copy(
          rows_v.at[b], out_hbm.at[bb, pl.ds(col + chunk * C, C)]
      )

    for b in range(NB):
      start_gather(b, b)

    def body(i, carry):
      for b in range(NB):
        chunk = i * NB + b
        wait_gather(chunk, b)
        put(chunk, b)
        start_gather(chunk + NB, b)
      return carry

    lax.fori_loop(0, NCHUNK // NB - 1, body, 0)

    for b in range(NB):
      chunk = NCHUNK - NB + b
      wait_gather(chunk, b)
      put(chunk, b)

  return k(weight, input_ids)


def kernel(input_ids, weight):
  return _sc_gather(weight, input_ids.astype(jnp.int32))


# speedup vs baseline: 1.0030x; 1.0030x over previous
import functools

import jax
import jax.numpy as jnp
from jax import lax
from jax.experimental import pallas as pl
from jax.experimental.pallas import tpu as pltpu
from jax.experimental.pallas import tpu_sc as plsc

BATCH = 4
SEQ = 8192
D = 1024

NC = 2
NS = 16
NW = NC * NS
B_PER_W = BATCH * SEQ // NW
W_PER_BATCH = SEQ // B_PER_W
C = 32
NCHUNK = B_PER_W // C
NB = 2


def _sc_gather(weight, input_ids):
  mesh = plsc.VectorSubcoreMesh(core_axis_name="c", subcore_axis_name="s")

  @functools.partial(
      pl.kernel,
      mesh=mesh,
      out_type=jax.ShapeDtypeStruct((BATCH, SEQ, D), jnp.float32),
      scratch_types=[
          pltpu.VMEM((B_PER_W,), jnp.int32),
          pltpu.VMEM((NB, C, D), jnp.float32),
          pltpu.SemaphoreType.DMA((NB,)),
      ],
  )
  def k(table_hbm, idx_hbm, out_hbm, idx_v, rows_v, gsem):
    wid = lax.axis_index("s") * NC + lax.axis_index("c")
    bb = wid // W_PER_BATCH
    col = (wid % W_PER_BATCH) * B_PER_W
    pltpu.sync_copy(idx_hbm.at[bb, pl.ds(col, B_PER_W)], idx_v)

    def start_gather(chunk, b):
      pltpu.async_copy(
          table_hbm.at[idx_v.at[pl.ds(chunk * C, C)]], rows_v.at[b], gsem.at[b]
      )

    def wait_gather(chunk, b):
      pltpu.make_async_copy(
          table_hbm.at[idx_v.at[pl.ds(chunk * C, C)]], rows_v.at[b], gsem.at[b]
      ).wait()

    def put(chunk, b):
      pltpu.sync_copy(
          rows_v.at[b], out_hbm.at[bb, pl.ds(col + chunk * C, C)]
      )

    for b in range(NB):
      start_gather(b, b)

    def body(i, carry):
      for b in range(NB):
        chunk = i * NB + b
        wait_gather(chunk, b)
        put(chunk, b)
        start_gather(chunk + NB, b)
      return carry

    lax.fori_loop(0, NCHUNK // NB - 1, body, 0)

    for b in range(NB):
      chunk = NCHUNK - NB + b
      wait_gather(chunk, b)
      put(chunk, b)

  return k(weight, input_ids)


def kernel(input_ids, weight):
  return _sc_gather(weight, input_ids.astype(jnp.int32))


# speedup vs baseline: 1.0039x; 1.0009x over previous
import functools

import jax
import jax.numpy as jnp
from jax import lax
from jax.experimental import pallas as pl
from jax.experimental.pallas import tpu as pltpu
from jax.experimental.pallas import tpu_sc as plsc

BATCH = 4
SEQ = 8192
D = 1024

NC = 2
NS = 16
NW = NC * NS
B_PER_W = BATCH * SEQ // NW
W_PER_BATCH = SEQ // B_PER_W
C = 16
NCHUNK = B_PER_W // C
NB = 4


def _sc_gather(weight, input_ids):
  mesh = plsc.VectorSubcoreMesh(core_axis_name="c", subcore_axis_name="s")

  @functools.partial(
      pl.kernel,
      mesh=mesh,
      out_type=jax.ShapeDtypeStruct((BATCH, SEQ, D), jnp.float32),
      scratch_types=[
          pltpu.VMEM((B_PER_W,), jnp.int32),
          pltpu.VMEM((NB, C, D), jnp.float32),
          pltpu.SemaphoreType.DMA((NB,)),
      ],
  )
  def k(table_hbm, idx_hbm, out_hbm, idx_v, rows_v, gsem):
    wid = lax.axis_index("s") * NC + lax.axis_index("c")
    bb = wid // W_PER_BATCH
    col = (wid % W_PER_BATCH) * B_PER_W
    pltpu.sync_copy(idx_hbm.at[bb, pl.ds(col, B_PER_W)], idx_v)

    def start_gather(chunk, b):
      pltpu.async_copy(
          table_hbm.at[idx_v.at[pl.ds(chunk * C, C)]], rows_v.at[b], gsem.at[b]
      )

    def wait_gather(chunk, b):
      pltpu.make_async_copy(
          table_hbm.at[idx_v.at[pl.ds(chunk * C, C)]], rows_v.at[b], gsem.at[b]
      ).wait()

    def put(chunk, b):
      pltpu.sync_copy(
          rows_v.at[b], out_hbm.at[bb, pl.ds(col + chunk * C, C)]
      )

    for b in range(NB):
      start_gather(b, b)

    def body(i, carry):
      for b in range(NB):
        chunk = i * NB + b
        wait_gather(chunk, b)
        put(chunk, b)
        start_gather(chunk + NB, b)
      return carry

    lax.fori_loop(0, NCHUNK // NB - 1, body, 0)

    for b in range(NB):
      chunk = NCHUNK - NB + b
      wait_gather(chunk, b)
      put(chunk, b)

  return k(weight, input_ids)


def kernel(input_ids, weight):
  return _sc_gather(weight, input_ids.astype(jnp.int32))


# speedup vs baseline: 1.0116x; 1.0076x over previous
import functools

import jax
import jax.numpy as jnp
from jax import lax
from jax.experimental import pallas as pl
from jax.experimental.pallas import tpu as pltpu
from jax.experimental.pallas import tpu_sc as plsc

BATCH = 4
SEQ = 8192
D = 1024

NC = 2
NS = 16
NW = NC * NS
B_PER_W = BATCH * SEQ // NW
W_PER_BATCH = SEQ // B_PER_W
C = 8
NCHUNK = B_PER_W // C
NB = 8


def _sc_gather(weight, input_ids):
  mesh = plsc.VectorSubcoreMesh(core_axis_name="c", subcore_axis_name="s")

  @functools.partial(
      pl.kernel,
      mesh=mesh,
      out_type=jax.ShapeDtypeStruct((BATCH, SEQ, D), jnp.float32),
      scratch_types=[
          pltpu.VMEM((B_PER_W,), jnp.int32),
          pltpu.VMEM((NB, C, D), jnp.float32),
          pltpu.SemaphoreType.DMA((NB,)),
      ],
  )
  def k(table_hbm, idx_hbm, out_hbm, idx_v, rows_v, gsem):
    wid = lax.axis_index("s") * NC + lax.axis_index("c")
    bb = wid // W_PER_BATCH
    col = (wid % W_PER_BATCH) * B_PER_W
    pltpu.sync_copy(idx_hbm.at[bb, pl.ds(col, B_PER_W)], idx_v)

    def start_gather(chunk, b):
      pltpu.async_copy(
          table_hbm.at[idx_v.at[pl.ds(chunk * C, C)]], rows_v.at[b], gsem.at[b]
      )

    def wait_gather(chunk, b):
      pltpu.make_async_copy(
          table_hbm.at[idx_v.at[pl.ds(chunk * C, C)]], rows_v.at[b], gsem.at[b]
      ).wait()

    def put(chunk, b):
      pltpu.sync_copy(
          rows_v.at[b], out_hbm.at[bb, pl.ds(col + chunk * C, C)]
      )

    for b in range(NB):
      start_gather(b, b)

    def body(i, carry):
      for b in range(NB):
        chunk = i * NB + b
        wait_gather(chunk, b)
        put(chunk, b)
        start_gather(chunk + NB, b)
      return carry

    lax.fori_loop(0, NCHUNK // NB - 1, body, 0)

    for b in range(NB):
      chunk = NCHUNK - NB + b
      wait_gather(chunk, b)
      put(chunk, b)

  return k(weight, input_ids)


def kernel(input_ids, weight):
  return _sc_gather(weight, input_ids.astype(jnp.int32))
